# Initial kernel scaffold; baseline (speedup 1.0000x reference)
import jax
import jax.numpy as jnp
from jax.experimental import pallas as pl


def kernel(x, pos, edge_attr, node_sigma_emb, W1, b1, W2, b2, We1, be1, We2, be2, Wtp, edge_index):
    raise NotImplementedError("write your pallas kernel here")


# speedup vs baseline: 1.8750x; 1.8750x over previous
import functools

import jax
import jax.numpy as jnp
from jax import lax
from jax.experimental import pallas as pl
from jax.experimental.pallas import tpu as pltpu
from jax.experimental.pallas import tpu_sc as plsc

NS = 16
SIGMA = 32
DIST = 32
NSH = 9
N_NODES = 50000
E_EDGES = 800000

NPAD = 51200
EPAD = 819200
CH = 128
NCHUNK = 200
PER_W = NCHUNK * CH
RPS = NPAD // 16
BE = 2048
BN = 3200

_COEFF = float(-0.5 / (5.0 / (DIST - 1)) ** 2)


def _prep_body(pos_ref, sig_ref, x_ref, w1b_ref, ts_ref, td_ref):
    p8 = pos_ref[...]
    s_emb = jnp.dot(sig_ref[...], w1b_ref[...],
                    preferred_element_type=jnp.float32)
    xb = x_ref[...]
    z8 = jnp.zeros((p8.shape[0], 8), jnp.float32)
    ts_ref[...] = jnp.concatenate([p8, s_emb, xb, z8], axis=1)
    td_ref[...] = jnp.concatenate([p8, xb, z8], axis=1)


def _prep_tables(pos8, sig, x, w1b):
    grid = (NPAD // BN,)
    return pl.pallas_call(
        _prep_body,
        grid=grid,
        in_specs=[
            pl.BlockSpec((BN, 8), lambda i: (i, 0)),
            pl.BlockSpec((BN, SIGMA), lambda i: (i, 0)),
            pl.BlockSpec((BN, NS), lambda i: (i, 0)),
            pl.BlockSpec((SIGMA, NS), lambda i: (0, 0)),
        ],
        out_specs=[
            pl.BlockSpec((BN, 48), lambda i: (i, 0)),
            pl.BlockSpec((BN, 32), lambda i: (i, 0)),
        ],
        out_shape=[
            jax.ShapeDtypeStruct((NPAD, 48), jnp.float32),
            jax.ShapeDtypeStruct((NPAD, 32), jnp.float32),
        ],
    )(pos8, sig, x, w1b)


def _edge0_body(gs_ref, gd_ref, ea_ref, offs_ref, w1a_ref, w1c_ref, b1_ref,
                w2_ref, b2_ref, we1_ref, be1_ref, we2_ref, be2_ref, wtp_ref,
                m_ref, es_ref):
    gs = gs_ref[...]
    gd = gd_ref[...]
    pv = gd[:, 0:3] - gs[:, 0:3]
    d2 = jnp.sum(pv * pv, axis=1, keepdims=True)
    d = jnp.sqrt(d2)
    demb = jnp.exp(_COEFF * (d - offs_ref[...]) ** 2)
    n = pv / (d + 1e-8)
    xc, yc, zc = n[:, 0:1], n[:, 1:2], n[:, 2:3]
    sh = [jnp.ones_like(xc), xc, yc, zc, xc * yc, yc * zc,
          3.0 * zc * zc - 1.0, xc * zc, xc * xc - yc * yc]
    e_lin = (jnp.dot(ea_ref[...], w1a_ref[...], preferred_element_type=jnp.float32)
             + gs[:, 8:24]
             + jnp.dot(demb, w1c_ref[...], preferred_element_type=jnp.float32)
             + b1_ref[...])
    e_emb = jnp.dot(jnp.maximum(e_lin, 0.0), w2_ref[...],
                    preferred_element_type=jnp.float32) + b2_ref[...]
    hs = gs[:, 24:40]
    hd = gd[:, 8:24]
    ef = jnp.concatenate([e_emb, hs, hd], axis=1)
    wv = jnp.dot(jnp.maximum(jnp.dot(ef, we1_ref[...],
                                     preferred_element_type=jnp.float32)
                             + be1_ref[...], 0.0),
                 we2_ref[...], preferred_element_type=jnp.float32) + be2_ref[...]
    t = hs * wv
    msg = sh[0] * jnp.dot(t, wtp_ref[0], preferred_element_type=jnp.float32)
    for s_i in range(1, NSH):
        msg = msg + sh[s_i] * jnp.dot(t, wtp_ref[s_i],
                                      preferred_element_type=jnp.float32)
    m_ref[...] = msg
    z7 = jnp.zeros((e_emb.shape[0], 7), jnp.float32)
    es_ref[...] = jnp.concatenate([e_emb] + sh + [z7], axis=1)


def _edge0(gs, gd, ea, offs, w1a, w1c, b1, w2, b2, we1, be1, we2, be2, wtp):
    grid = (EPAD // BE,)
    full = lambda *s: pl.BlockSpec(s, lambda i: tuple(0 for _ in s))
    return pl.pallas_call(
        _edge0_body,
        grid=grid,
        in_specs=[
            pl.BlockSpec((BE, 48), lambda i: (i, 0)),
            pl.BlockSpec((BE, 32), lambda i: (i, 0)),
            pl.BlockSpec((BE, 4), lambda i: (i, 0)),
            full(1, DIST),
            full(4, NS),
            full(DIST, NS),
            full(1, NS),
            full(NS, NS),
            full(1, NS),
            full(3 * NS, 3 * NS),
            full(1, 3 * NS),
            full(3 * NS, NS),
            full(1, NS),
            full(NSH, NS, NS),
        ],
        out_specs=[
            pl.BlockSpec((BE, NS), lambda i: (i, 0)),
            pl.BlockSpec((BE, 32), lambda i: (i, 0)),
        ],
        out_shape=[
            jax.ShapeDtypeStruct((EPAD, NS), jnp.float32),
            jax.ShapeDtypeStruct((EPAD, 32), jnp.float32),
        ],
    )(gs, gd, ea, offs, w1a, w1c, b1, w2, b2, we1, be1, we2, be2, wtp)


def _edge1_body(es_ref, hs_ref, hd_ref, we1_ref, be1_ref, we2_ref, be2_ref,
                wtp_ref, m_ref):
    es = es_ref[...]
    e_emb = es[:, 0:NS]
    hs = hs_ref[...]
    hd = hd_ref[...]
    ef = jnp.concatenate([e_emb, hs, hd], axis=1)
    wv = jnp.dot(jnp.maximum(jnp.dot(ef, we1_ref[...],
                                     preferred_element_type=jnp.float32)
                             + be1_ref[...], 0.0),
                 we2_ref[...], preferred_element_type=jnp.float32) + be2_ref[...]
    t = hs * wv
    msg = es[:, NS:NS + 1] * jnp.dot(t, wtp_ref[0],
                                     preferred_element_type=jnp.float32)
    for s_i in range(1, NSH):
        msg = msg + es[:, NS + s_i:NS + s_i + 1] * jnp.dot(
            t, wtp_ref[s_i], preferred_element_type=jnp.float32)
    m_ref[...] = msg


def _edge1(es, hs, hd, we1, be1, we2, be2, wtp):
    grid = (EPAD // BE,)
    full = lambda *s: pl.BlockSpec(s, lambda i: tuple(0 for _ in s))
    return pl.pallas_call(
        _edge1_body,
        grid=grid,
        in_specs=[
            pl.BlockSpec((BE, 32), lambda i: (i, 0)),
            pl.BlockSpec((BE, NS), lambda i: (i, 0)),
            pl.BlockSpec((BE, NS), lambda i: (i, 0)),
            full(3 * NS, 3 * NS),
            full(1, 3 * NS),
            full(3 * NS, NS),
            full(1, NS),
            full(NSH, NS, NS),
        ],
        out_specs=pl.BlockSpec((BE, NS), lambda i: (i, 0)),
        out_shape=jax.ShapeDtypeStruct((EPAD, NS), jnp.float32),
    )(es, hs, hd, we1, be1, we2, be2, wtp)


def _combine0_body(p_ref, d_ref, h_ref, deg_ref):
    ps = p_ref[0] + p_ref[1]
    ds_ = d_ref[0] + d_ref[1]
    degc = jnp.maximum(ds_, 1.0)
    h_ref[...] = ps / degc
    deg_ref[...] = degc


def _combine0(p, dacc):
    grid = (NPAD // BN,)
    return pl.pallas_call(
        _combine0_body,
        grid=grid,
        in_specs=[
            pl.BlockSpec((2, BN, NS), lambda i: (0, i, 0)),
            pl.BlockSpec((2, BN, NS), lambda i: (0, i, 0)),
        ],
        out_specs=[
            pl.BlockSpec((BN, NS), lambda i: (i, 0)),
            pl.BlockSpec((BN, NS), lambda i: (i, 0)),
        ],
        out_shape=[
            jax.ShapeDtypeStruct((NPAD, NS), jnp.float32),
            jax.ShapeDtypeStruct((NPAD, NS), jnp.float32),
        ],
    )(p, dacc)


def _combine1_body(p_ref, deg_ref, h_ref):
    h_ref[...] = (p_ref[0] + p_ref[1]) / deg_ref[...]


def _combine1(p, degc):
    grid = (NPAD // BN,)
    return pl.pallas_call(
        _combine1_body,
        grid=grid,
        in_specs=[
            pl.BlockSpec((2, BN, NS), lambda i: (0, i, 0)),
            pl.BlockSpec((BN, NS), lambda i: (i, 0)),
        ],
        out_specs=pl.BlockSpec((BN, NS), lambda i: (i, 0)),
        out_shape=jax.ShapeDtypeStruct((NPAD, NS), jnp.float32),
    )(p, degc)


_MESH = dict(core_axis_name="c", subcore_axis_name="s")


def _sc_gather_ab(ts, td, si2, di2):
    mesh = plsc.VectorSubcoreMesh(**_MESH)

    @functools.partial(
        pl.kernel, mesh=mesh,
        compiler_params=pltpu.CompilerParams(use_tc_tiling_on_sc=False),
        out_type=[jax.ShapeDtypeStruct((EPAD, 48), jnp.float32),
                  jax.ShapeDtypeStruct((EPAD, 32), jnp.float32)],
        scratch_types=[pltpu.VMEM((NCHUNK, CH), jnp.int32),
                       pltpu.VMEM((NCHUNK, CH), jnp.int32),
                       pltpu.VMEM((CH, 48), jnp.float32),
                       pltpu.VMEM((CH, 32), jnp.float32),
                       pltpu.SemaphoreType.DMA],
    )
    def k(ts_h, td_h, si_h, di_h, gs_h, gd_h, si_v, di_v, rs_v, rd_v, sem):
        w = lax.axis_index("c") * 16 + lax.axis_index("s")
        cb = w * NCHUNK
        pltpu.sync_copy(si_h.at[pl.ds(cb, NCHUNK)], si_v)
        pltpu.sync_copy(di_h.at[pl.ds(cb, NCHUNK)], di_v)

        def body(j, carry):
            off = (cb + j) * CH
            pltpu.async_copy(ts_h.at[si_v.at[j]], rs_v, sem).wait()
            pltpu.sync_copy(rs_v, gs_h.at[pl.ds(off, CH)])
            pltpu.async_copy(td_h.at[di_v.at[j]], rd_v, sem).wait()
            pltpu.sync_copy(rd_v, gd_h.at[pl.ds(off, CH)])
            return carry

        lax.fori_loop(0, NCHUNK, body, 0)

    return k(ts, td, si2, di2)


def _sc_gather_h(h, si2, di2):
    mesh = plsc.VectorSubcoreMesh(**_MESH)

    @functools.partial(
        pl.kernel, mesh=mesh,
        compiler_params=pltpu.CompilerParams(use_tc_tiling_on_sc=False),
        out_type=[jax.ShapeDtypeStruct((EPAD, NS), jnp.float32),
                  jax.ShapeDtypeStruct((EPAD, NS), jnp.float32)],
        scratch_types=[pltpu.VMEM((NCHUNK, CH), jnp.int32),
                       pltpu.VMEM((NCHUNK, CH), jnp.int32),
                       pltpu.VMEM((CH, NS), jnp.float32),
                       pltpu.VMEM((CH, NS), jnp.float32),
                       pltpu.SemaphoreType.DMA],
    )
    def k(h_h, si_h, di_h, hs_h, hd_h, si_v, di_v, rs_v, rd_v, sem):
        w = lax.axis_index("c") * 16 + lax.axis_index("s")
        cb = w * NCHUNK
        pltpu.sync_copy(si_h.at[pl.ds(cb, NCHUNK)], si_v)
        pltpu.sync_copy(di_h.at[pl.ds(cb, NCHUNK)], di_v)

        def body(j, carry):
            off = (cb + j) * CH
            pltpu.async_copy(h_h.at[si_v.at[j]], rs_v, sem).wait()
            pltpu.sync_copy(rs_v, hs_h.at[pl.ds(off, CH)])
            pltpu.async_copy(h_h.at[di_v.at[j]], rd_v, sem).wait()
            pltpu.sync_copy(rd_v, hd_h.at[pl.ds(off, CH)])
            return carry

        lax.fori_loop(0, NCHUNK, body, 0)

    return k(h, si2, di2)


def _sc_scatter_deg(di2, msg):
    mesh = plsc.VectorSubcoreMesh(**_MESH)

    @functools.partial(
        pl.kernel, mesh=mesh,
        compiler_params=pltpu.CompilerParams(use_tc_tiling_on_sc=False),
        out_type=[jax.ShapeDtypeStruct((2, NPAD, NS), jnp.float32),
                  jax.ShapeDtypeStruct((2, NPAD, NS), jnp.float32)],
        scratch_types=[pltpu.VMEM_SHARED((NPAD, NS), jnp.float32),
                       pltpu.VMEM_SHARED((NPAD, NS), jnp.float32),
                       pltpu.VMEM((25, CH), jnp.int32),
                       pltpu.VMEM((CH, NS), jnp.float32),
                       pltpu.VMEM((CH, NS), jnp.float32)],
    )
    def k(di_h, m_h, p_h, dg_h, accm, accd, di_v, rows_v, fill_v):
        c = lax.axis_index("c")
        s = lax.axis_index("s")
        w = c * 16 + s
        cb = w * NCHUNK

        def zb(i, carry):
            fill_v[i, :] = jnp.zeros((NS,), jnp.float32)
            return carry
        lax.fori_loop(0, CH, zb, 0)

        def zc(kk, carry):
            pltpu.sync_copy(fill_v, accm.at[pl.ds(s * RPS + kk * CH, CH)])
            pltpu.sync_copy(fill_v, accd.at[pl.ds(s * RPS + kk * CH, CH)])
            return carry
        lax.fori_loop(0, RPS // CH, zc, 0)
        plsc.subcore_barrier()

        def ob(i, carry):
            fill_v[i, :] = jnp.ones((NS,), jnp.float32)
            return carry
        lax.fori_loop(0, CH, ob, 0)

        def outer(jo, carry):
            pltpu.sync_copy(di_h.at[pl.ds(cb + jo * 25, 25)], di_v)

            def body(j, carry2):
                off = (cb + jo * 25 + j) * CH
                pltpu.sync_copy(m_h.at[pl.ds(off, CH)], rows_v)
                pltpu.sync_copy(rows_v, accm.at[di_v.at[j]], add=True)
                pltpu.sync_copy(fill_v, accd.at[di_v.at[j]], add=True)
                return carry2
            lax.fori_loop(0, 25, body, 0)
            return carry
        lax.fori_loop(0, NCHUNK // 25, outer, 0)
        plsc.subcore_barrier()

        pltpu.sync_copy(accm.at[pl.ds(s * RPS, RPS)],
                        p_h.at[c, pl.ds(s * RPS, RPS)])
        pltpu.sync_copy(accd.at[pl.ds(s * RPS, RPS)],
                        dg_h.at[c, pl.ds(s * RPS, RPS)])

    return k(di2, msg)


def _sc_scatter(di2, msg):
    mesh = plsc.VectorSubcoreMesh(**_MESH)

    @functools.partial(
        pl.kernel, mesh=mesh,
        compiler_params=pltpu.CompilerParams(use_tc_tiling_on_sc=False),
        out_type=jax.ShapeDtypeStruct((2, NPAD, NS), jnp.float32),
        scratch_types=[pltpu.VMEM_SHARED((NPAD, NS), jnp.float32),
                       pltpu.VMEM((25, CH), jnp.int32),
                       pltpu.VMEM((CH, NS), jnp.float32),
                       pltpu.VMEM((CH, NS), jnp.float32)],
    )
    def k(di_h, m_h, p_h, accm, di_v, rows_v, fill_v):
        c = lax.axis_index("c")
        s = lax.axis_index("s")
        w = c * 16 + s
        cb = w * NCHUNK

        def zb(i, carry):
            fill_v[i, :] = jnp.zeros((NS,), jnp.float32)
            return carry
        lax.fori_loop(0, CH, zb, 0)

        def zc(kk, carry):
            pltpu.sync_copy(fill_v, accm.at[pl.ds(s * RPS + kk * CH, CH)])
            return carry
        lax.fori_loop(0, RPS // CH, zc, 0)
        plsc.subcore_barrier()

        def outer(jo, carry):
            pltpu.sync_copy(di_h.at[pl.ds(cb + jo * 25, 25)], di_v)

            def body(j, carry2):
                off = (cb + jo * 25 + j) * CH
                pltpu.sync_copy(m_h.at[pl.ds(off, CH)], rows_v)
                pltpu.sync_copy(rows_v, accm.at[di_v.at[j]], add=True)
                return carry2
            lax.fori_loop(0, 25, body, 0)
            return carry
        lax.fori_loop(0, NCHUNK // 25, outer, 0)
        plsc.subcore_barrier()

        pltpu.sync_copy(accm.at[pl.ds(s * RPS, RPS)],
                        p_h.at[c, pl.ds(s * RPS, RPS)])

    return k(di2, msg)


def kernel(x, pos, edge_attr, node_sigma_emb, W1, b1, W2, b2, We1, be1,
           We2, be2, Wtp, edge_index):
    f32 = jnp.float32
    pos8 = jnp.pad(pos.astype(f32), ((0, NPAD - N_NODES), (0, 5)))
    sig = jnp.pad(node_sigma_emb, ((0, NPAD - N_NODES), (0, 0)))
    x_p = jnp.pad(x, ((0, NPAD - N_NODES), (0, 0)))
    ea = jnp.pad(edge_attr, ((0, EPAD - E_EDGES), (0, 0)))
    si2 = jnp.pad(edge_index[0], (0, EPAD - E_EDGES)).reshape(EPAD // CH, CH)
    di2 = jnp.pad(edge_index[1], (0, EPAD - E_EDGES),
                  constant_values=N_NODES).reshape(EPAD // CH, CH)
    w1a = W1[0:4]
    w1b = W1[4:4 + SIGMA]
    w1c = W1[4 + SIGMA:]
    offs = jnp.linspace(0.0, 5.0, DIST).astype(f32).reshape(1, DIST)
    wtpT = jnp.transpose(Wtp, (0, 2, 1, 3))
    b1r = b1.reshape(1, NS)
    b2r = b2.reshape(1, NS)

    ts, td = _prep_tables(pos8, sig, x_p, w1b)
    gs, gd = _sc_gather_ab(ts, td, si2, di2)
    m0, es = _edge0(gs, gd, ea, offs, w1a, w1c, b1r, W2, b2r,
                    We1[0], be1[0].reshape(1, 3 * NS), We2[0],
                    be2[0].reshape(1, NS), wtpT[0])
    p0, dacc = _sc_scatter_deg(di2, m0)
    h1, degc = _combine0(p0, dacc)
    hs1, hd1 = _sc_gather_h(h1, si2, di2)
    m1 = _edge1(es, hs1, hd1, We1[1], be1[1].reshape(1, 3 * NS), We2[1],
                be2[1].reshape(1, NS), wtpT[1])
    p1 = _sc_scatter(di2, m1)
    h2 = _combine1(p1, degc)
    return h2[:N_NODES]


# speedup vs baseline: 2.4005x; 1.2803x over previous
import functools

import numpy as np

import jax
import jax.numpy as jnp
from jax import lax
from jax.experimental import pallas as pl
from jax.experimental.pallas import tpu as pltpu
from jax.experimental.pallas import tpu_sc as plsc

NS = 16
SIGMA = 32
DIST = 32
NSH = 9
N_NODES = 50000
E_EDGES = 800000

NPAD = 51200
EPAD = 819200
CH = 128
NCHUNK = 200
RPS = NPAD // 16
BE = 2048
BN = 3200

_COEFF = float(-0.5 / (5.0 / (DIST - 1)) ** 2)

_R16_NP = np.zeros((NS, NSH * NS), np.float32)
for _s in range(NSH):
    _R16_NP[_s, _s * NS:(_s + 1) * NS] = 1.0
_R32_NP = np.zeros((2 * NS, NSH * NS), np.float32)
_R32_NP[NS:, :] = _R16_NP
_G_NP = np.zeros((NSH * NS, NS), np.float32)
for _s in range(NSH):
    _G_NP[_s * NS:(_s + 1) * NS, :] = np.eye(NS, dtype=np.float32)

_PA_NP = np.zeros((NS, NS), np.float32)
_PB_NP = np.zeros((NS, NS), np.float32)
_CA_NP = np.zeros((1, NS), np.float32)
_CB_NP = np.zeros((1, NS), np.float32)
_CC_NP = np.zeros((1, NS), np.float32)
_CA_NP[0, 0] = 1.0
_CB_NP[0, 0:4] = 1.0
_CC_NP[0, 6] = -1.0
_PA_NP[0, 1] = 1.0
_PA_NP[1, 2] = 1.0
_PA_NP[2, 3] = 1.0
_PA_NP[0, 4] = 1.0
_PB_NP[1, 4] = 1.0
_PA_NP[1, 5] = 1.0
_PB_NP[2, 5] = 1.0
_PA_NP[2, 6] = 3.0
_PB_NP[2, 6] = 1.0
_PA_NP[0, 7] = 1.0
_PB_NP[2, 7] = 1.0
_PA_NP[0, 8] = 1.0
_PA_NP[1, 8] = -1.0
_PB_NP[0, 8] = 1.0
_PB_NP[1, 8] = 1.0
_J16_NP = np.ones((NS, NS), np.float32)
_J32_NP = np.ones((NS, DIST), np.float32)


def _prep_body(sig_ref, w1b_ref, s16_ref):
    s16_ref[...] = jnp.dot(sig_ref[...], w1b_ref[...],
                           preferred_element_type=jnp.float32)


def _prep_s16(sig, w1b):
    return pl.pallas_call(
        _prep_body,
        grid=(NPAD // BN,),
        in_specs=[
            pl.BlockSpec((BN, SIGMA), lambda i: (i, 0)),
            pl.BlockSpec((SIGMA, NS), lambda i: (0, 0)),
        ],
        out_specs=pl.BlockSpec((BN, NS), lambda i: (i, 0)),
        out_shape=jax.ShapeDtypeStruct((NPAD, NS), jnp.float32),
    )(sig, w1b)


def _edge0_body(ps_ref, pd_ref, ss_ref, xs_ref, xd_ref, ea_ref, offs_ref,
                w1a_ref, w1c_ref, b1_ref, w2_ref, b2_ref, we1_ref, be1_ref,
                we2_ref, be2_ref, wtp_ref, r_ref, g_ref, pa_ref, pb_ref,
                ca_ref, cb_ref, cc_ref, j16_ref, j32_ref, m_ref, es_ref):
    f32 = jnp.float32
    pv = pd_ref[...] - ps_ref[...]
    q = pv * pv
    d2_16 = jnp.dot(q, j16_ref[...], preferred_element_type=f32)
    d2_32 = jnp.dot(q, j32_ref[...], preferred_element_type=f32)
    d32 = jnp.sqrt(d2_32)
    demb = jnp.exp(_COEFF * (d32 - offs_ref[...]) ** 2)
    n16 = pv / (jnp.sqrt(d2_16) + 1e-8)
    sh16 = ((jnp.dot(n16, pa_ref[...], preferred_element_type=f32)
             + ca_ref[...])
            * (jnp.dot(n16, pb_ref[...], preferred_element_type=f32)
               + cb_ref[...])
            + cc_ref[...])
    e_lin = (jnp.dot(ea_ref[...], w1a_ref[...], preferred_element_type=f32)
             + ss_ref[...]
             + jnp.dot(demb, w1c_ref[...], preferred_element_type=f32)
             + b1_ref[...])
    e_emb = jnp.dot(jnp.maximum(e_lin, 0.0), w2_ref[...],
                    preferred_element_type=f32) + b2_ref[...]
    hs = xs_ref[...]
    hd = xd_ref[...]
    ef = jnp.concatenate([e_emb, hs, hd], axis=1)
    wv = jnp.dot(jnp.maximum(jnp.dot(ef, we1_ref[...],
                                     preferred_element_type=f32)
                             + be1_ref[...], 0.0),
                 we2_ref[...], preferred_element_type=f32) + be2_ref[...]
    t = hs * wv
    shrep = jnp.dot(sh16, r_ref[...], preferred_element_type=f32)
    tmp = jnp.dot(t, wtp_ref[...], preferred_element_type=f32)
    m_ref[...] = jnp.dot(shrep * tmp, g_ref[...],
                         preferred_element_type=f32)
    es_ref[...] = jnp.concatenate([e_emb, sh16], axis=1)


def _edge0(ps, pd, ss, xs, xd, ea, offs, w1a, w1c, b1, w2, b2,
           we1, be1, we2, be2, wcat, rmat, gmat, pa, pb, ca, cb, cc,
           j16, j32):
    full = lambda *s: pl.BlockSpec(s, lambda i: tuple(0 for _ in s))
    eb = lambda c: pl.BlockSpec((BE, c), lambda i: (i, 0))
    return pl.pallas_call(
        _edge0_body,
        grid=(EPAD // BE,),
        in_specs=[
            eb(NS), eb(NS), eb(NS), eb(NS), eb(NS), eb(4),
            full(1, DIST),
            full(4, NS),
            full(DIST, NS),
            full(1, NS),
            full(NS, NS),
            full(1, NS),
            full(3 * NS, 3 * NS),
            full(1, 3 * NS),
            full(3 * NS, NS),
            full(1, NS),
            full(NS, NSH * NS),
            full(NS, NSH * NS),
            full(NSH * NS, NS),
            full(NS, NS),
            full(NS, NS),
            full(1, NS),
            full(1, NS),
            full(1, NS),
            full(NS, NS),
            full(NS, DIST),
        ],
        out_specs=[eb(NS), eb(32)],
        out_shape=[
            jax.ShapeDtypeStruct((EPAD, NS), jnp.float32),
            jax.ShapeDtypeStruct((EPAD, 32), jnp.float32),
        ],
    )(ps, pd, ss, xs, xd, ea, offs, w1a, w1c, b1, w2, b2,
      we1, be1, we2, be2, wcat, rmat, gmat, pa, pb, ca, cb, cc, j16, j32)


def _edge1_body(es_ref, hs_ref, hd_ref, we1_ref, be1_ref, we2_ref, be2_ref,
                wtp_ref, r_ref, g_ref, m_ref):
    es = es_ref[...]
    e_emb = es[:, 0:NS]
    hs = hs_ref[...]
    hd = hd_ref[...]
    ef = jnp.concatenate([e_emb, hs, hd], axis=1)
    wv = jnp.dot(jnp.maximum(jnp.dot(ef, we1_ref[...],
                                     preferred_element_type=jnp.float32)
                             + be1_ref[...], 0.0),
                 we2_ref[...], preferred_element_type=jnp.float32) + be2_ref[...]
    t = hs * wv
    shrep = jnp.dot(es, r_ref[...], preferred_element_type=jnp.float32)
    tmp = jnp.dot(t, wtp_ref[...], preferred_element_type=jnp.float32)
    m_ref[...] = jnp.dot(shrep * tmp, g_ref[...],
                         preferred_element_type=jnp.float32)


def _edge1(es, hs, hd, we1, be1, we2, be2, wcat, rmat, gmat):
    full = lambda *s: pl.BlockSpec(s, lambda i: tuple(0 for _ in s))
    eb = lambda c: pl.BlockSpec((BE, c), lambda i: (i, 0))
    return pl.pallas_call(
        _edge1_body,
        grid=(EPAD // BE,),
        in_specs=[
            eb(32), eb(NS), eb(NS),
            full(3 * NS, 3 * NS),
            full(1, 3 * NS),
            full(3 * NS, NS),
            full(1, NS),
            full(NS, NSH * NS),
            full(2 * NS, NSH * NS),
            full(NSH * NS, NS),
        ],
        out_specs=eb(NS),
        out_shape=jax.ShapeDtypeStruct((EPAD, NS), jnp.float32),
    )(es, hs, hd, we1, be1, we2, be2, wcat, rmat, gmat)


def _combine0_body(p_ref, d_ref, h_ref, deg_ref):
    degc = jnp.maximum(d_ref[0] + d_ref[1], 1.0)
    h_ref[...] = (p_ref[0] + p_ref[1]) / degc
    deg_ref[...] = degc


def _combine0(p, dacc):
    return pl.pallas_call(
        _combine0_body,
        grid=(NPAD // BN,),
        in_specs=[
            pl.BlockSpec((2, BN, NS), lambda i: (0, i, 0)),
            pl.BlockSpec((2, BN, NS), lambda i: (0, i, 0)),
        ],
        out_specs=[
            pl.BlockSpec((BN, NS), lambda i: (i, 0)),
            pl.BlockSpec((BN, NS), lambda i: (i, 0)),
        ],
        out_shape=[
            jax.ShapeDtypeStruct((NPAD, NS), jnp.float32),
            jax.ShapeDtypeStruct((NPAD, NS), jnp.float32),
        ],
    )(p, dacc)


def _combine1_body(p_ref, deg_ref, h_ref):
    h_ref[...] = (p_ref[0] + p_ref[1]) / deg_ref[...]


def _combine1(p, degc):
    return pl.pallas_call(
        _combine1_body,
        grid=(NPAD // BN,),
        in_specs=[
            pl.BlockSpec((2, BN, NS), lambda i: (0, i, 0)),
            pl.BlockSpec((BN, NS), lambda i: (i, 0)),
        ],
        out_specs=pl.BlockSpec((BN, NS), lambda i: (i, 0)),
        out_shape=jax.ShapeDtypeStruct((NPAD, NS), jnp.float32),
    )(p, degc)


_MESH = dict(core_axis_name="c", subcore_axis_name="s")


def _gather_pipeline(streams, si_h, di_h, si_v, di_v, sems, cb):
    semA, semB = sems
    pltpu.sync_copy(si_h.at[pl.ds(cb, NCHUNK)], si_v)
    pltpu.sync_copy(di_h.at[pl.ds(cb, NCHUNK)], di_v)
    idxs = {0: si_v, 1: di_v}

    def issue(j, par, sem):
        for tbl, wi, _, buf in streams:
            pltpu.async_copy(tbl.at[idxs[wi].at[j]],
                             buf.at[pl.ds(par, CH)], sem)

    def drain_write(j, par, sem):
        for tbl, wi, out, buf in streams:
            pltpu.make_async_copy(out.at[pl.ds(0, CH)],
                                  buf.at[pl.ds(par, CH)], sem).wait()
        off = (cb + j) * CH
        for _, _, out, buf in streams:
            pltpu.sync_copy(buf.at[pl.ds(par, CH)], out.at[pl.ds(off, CH)])

    issue(0, 0, semA)

    def outer(k, carry):
        j0 = 2 * k
        issue(j0 + 1, CH, semB)
        drain_write(j0, 0, semA)

        @pl.when(k < NCHUNK // 2 - 1)
        def _():
            issue(j0 + 2, 0, semA)

        drain_write(j0 + 1, CH, semB)
        return carry

    lax.fori_loop(0, NCHUNK // 2, outer, 0)


def _sc_gather_ab(pos16, s16, x16, si2, di2):
    mesh = plsc.VectorSubcoreMesh(**_MESH)
    o16 = jax.ShapeDtypeStruct((EPAD, NS), jnp.float32)
    buf = pltpu.VMEM((2 * CH, NS), jnp.float32)

    @functools.partial(
        pl.kernel, mesh=mesh,
        compiler_params=pltpu.CompilerParams(use_tc_tiling_on_sc=False),
        out_type=[o16, o16, o16, o16, o16],
        scratch_types=[pltpu.VMEM((NCHUNK, CH), jnp.int32),
                       pltpu.VMEM((NCHUNK, CH), jnp.int32),
                       buf, buf, buf, buf, buf,
                       pltpu.SemaphoreType.DMA,
                       pltpu.SemaphoreType.DMA],
    )
    def k(p_h, s_h, x_h, si_h, di_h, ps_h, ss_h, xs_h, pd_h, xd_h,
          si_v, di_v, b0, b1, b2, b3, b4, semA, semB):
        w = lax.axis_index("c") * 16 + lax.axis_index("s")
        cb = w * NCHUNK
        streams = [(p_h, 0, ps_h, b0), (s_h, 0, ss_h, b1), (x_h, 0, xs_h, b2),
                   (p_h, 1, pd_h, b3), (x_h, 1, xd_h, b4)]
        _gather_pipeline(streams, si_h, di_h, si_v, di_v, (semA, semB), cb)

    return k(pos16, s16, x16, si2, di2)


def _sc_gather_h(h, si2, di2):
    mesh = plsc.VectorSubcoreMesh(**_MESH)
    o16 = jax.ShapeDtypeStruct((EPAD, NS), jnp.float32)
    buf = pltpu.VMEM((2 * CH, NS), jnp.float32)

    @functools.partial(
        pl.kernel, mesh=mesh,
        compiler_params=pltpu.CompilerParams(use_tc_tiling_on_sc=False),
        out_type=[o16, o16],
        scratch_types=[pltpu.VMEM((NCHUNK, CH), jnp.int32),
                       pltpu.VMEM((NCHUNK, CH), jnp.int32),
                       buf, buf,
                       pltpu.SemaphoreType.DMA,
                       pltpu.SemaphoreType.DMA],
    )
    def k(h_h, si_h, di_h, hs_h, hd_h, si_v, di_v, b0, b1, semA, semB):
        w = lax.axis_index("c") * 16 + lax.axis_index("s")
        cb = w * NCHUNK
        streams = [(h_h, 0, hs_h, b0), (h_h, 1, hd_h, b1)]
        _gather_pipeline(streams, si_h, di_h, si_v, di_v, (semA, semB), cb)

    return k(h, si2, di2)


def _sc_scatter_deg(di2, msg):
    mesh = plsc.VectorSubcoreMesh(**_MESH)

    @functools.partial(
        pl.kernel, mesh=mesh,
        compiler_params=pltpu.CompilerParams(use_tc_tiling_on_sc=False),
        out_type=[jax.ShapeDtypeStruct((2, NPAD, NS), jnp.float32),
                  jax.ShapeDtypeStruct((2, NPAD, NS), jnp.float32)],
        scratch_types=[pltpu.VMEM_SHARED((NPAD, NS), jnp.float32),
                       pltpu.VMEM_SHARED((NPAD, NS), jnp.float32),
                       pltpu.VMEM((25, CH), jnp.int32),
                       pltpu.VMEM((CH, NS), jnp.float32),
                       pltpu.VMEM((CH, NS), jnp.float32)],
    )
    def k(di_h, m_h, p_h, dg_h, accm, accd, di_v, rows_v, fill_v):
        c = lax.axis_index("c")
        s = lax.axis_index("s")
        w = c * 16 + s
        cb = w * NCHUNK

        def zb(i, carry):
            fill_v[i, :] = jnp.zeros((NS,), jnp.float32)
            return carry
        lax.fori_loop(0, CH, zb, 0)

        def zc(kk, carry):
            pltpu.sync_copy(fill_v, accm.at[pl.ds(s * RPS + kk * CH, CH)])
            pltpu.sync_copy(fill_v, accd.at[pl.ds(s * RPS + kk * CH, CH)])
            return carry
        lax.fori_loop(0, RPS // CH, zc, 0)
        plsc.subcore_barrier()

        def ob(i, carry):
            fill_v[i, :] = jnp.ones((NS,), jnp.float32)
            return carry
        lax.fori_loop(0, CH, ob, 0)

        def outer(jo, carry):
            pltpu.sync_copy(di_h.at[pl.ds(cb + jo * 25, 25)], di_v)

            def body(j, carry2):
                off = (cb + jo * 25 + j) * CH
                pltpu.sync_copy(m_h.at[pl.ds(off, CH)], rows_v)
                pltpu.sync_copy(rows_v, accm.at[di_v.at[j]], add=True)
                pltpu.sync_copy(fill_v, accd.at[di_v.at[j]], add=True)
                return carry2
            lax.fori_loop(0, 25, body, 0)
            return carry
        lax.fori_loop(0, NCHUNK // 25, outer, 0)
        plsc.subcore_barrier()

        pltpu.sync_copy(accm.at[pl.ds(s * RPS, RPS)],
                        p_h.at[c, pl.ds(s * RPS, RPS)])
        pltpu.sync_copy(accd.at[pl.ds(s * RPS, RPS)],
                        dg_h.at[c, pl.ds(s * RPS, RPS)])

    return k(di2, msg)


def _sc_scatter(di2, msg):
    mesh = plsc.VectorSubcoreMesh(**_MESH)

    @functools.partial(
        pl.kernel, mesh=mesh,
        compiler_params=pltpu.CompilerParams(use_tc_tiling_on_sc=False),
        out_type=jax.ShapeDtypeStruct((2, NPAD, NS), jnp.float32),
        scratch_types=[pltpu.VMEM_SHARED((NPAD, NS), jnp.float32),
                       pltpu.VMEM((25, CH), jnp.int32),
                       pltpu.VMEM((CH, NS), jnp.float32),
                       pltpu.VMEM((CH, NS), jnp.float32)],
    )
    def k(di_h, m_h, p_h, accm, di_v, rows_v, fill_v):
        c = lax.axis_index("c")
        s = lax.axis_index("s")
        w = c * 16 + s
        cb = w * NCHUNK

        def zb(i, carry):
            fill_v[i, :] = jnp.zeros((NS,), jnp.float32)
            return carry
        lax.fori_loop(0, CH, zb, 0)

        def zc(kk, carry):
            pltpu.sync_copy(fill_v, accm.at[pl.ds(s * RPS + kk * CH, CH)])
            return carry
        lax.fori_loop(0, RPS // CH, zc, 0)
        plsc.subcore_barrier()

        def outer(jo, carry):
            pltpu.sync_copy(di_h.at[pl.ds(cb + jo * 25, 25)], di_v)

            def body(j, carry2):
                off = (cb + jo * 25 + j) * CH
                pltpu.sync_copy(m_h.at[pl.ds(off, CH)], rows_v)
                pltpu.sync_copy(rows_v, accm.at[di_v.at[j]], add=True)
                return carry2
            lax.fori_loop(0, 25, body, 0)
            return carry
        lax.fori_loop(0, NCHUNK // 25, outer, 0)
        plsc.subcore_barrier()

        pltpu.sync_copy(accm.at[pl.ds(s * RPS, RPS)],
                        p_h.at[c, pl.ds(s * RPS, RPS)])

    return k(di2, msg)


def kernel(x, pos, edge_attr, node_sigma_emb, W1, b1, W2, b2, We1, be1,
           We2, be2, Wtp, edge_index):
    f32 = jnp.float32
    pos16 = jnp.pad(pos.astype(f32), ((0, NPAD - N_NODES), (0, NS - 3)))
    x16 = jnp.pad(x, ((0, NPAD - N_NODES), (0, 0)))
    sig = jnp.pad(node_sigma_emb, ((0, NPAD - N_NODES), (0, 0)))
    ea = jnp.pad(edge_attr, ((0, EPAD - E_EDGES), (0, 0)))
    si2 = jnp.pad(edge_index[0], (0, EPAD - E_EDGES)).reshape(EPAD // CH, CH)
    di2 = jnp.pad(edge_index[1], (0, EPAD - E_EDGES),
                  constant_values=N_NODES).reshape(EPAD // CH, CH)
    w1a = W1[0:4]
    w1b = W1[4:4 + SIGMA]
    w1c = W1[4 + SIGMA:]
    offs = jnp.linspace(0.0, 5.0, DIST).astype(f32).reshape(1, DIST)
    wcat = Wtp.reshape(2, NS, NSH * NS)
    r16 = jnp.asarray(_R16_NP)
    r32 = jnp.asarray(_R32_NP)
    gmat = jnp.asarray(_G_NP)
    pa = jnp.asarray(_PA_NP)
    pb = jnp.asarray(_PB_NP)
    ca = jnp.asarray(_CA_NP)
    cb = jnp.asarray(_CB_NP)
    cc = jnp.asarray(_CC_NP)
    j16 = jnp.asarray(_J16_NP)
    j32 = jnp.asarray(_J32_NP)
    b1r = b1.reshape(1, NS)
    b2r = b2.reshape(1, NS)

    s16 = _prep_s16(sig, w1b)
    ps, ss, xs, pd, xd = _sc_gather_ab(pos16, s16, x16, si2, di2)
    m0, es = _edge0(ps, pd, ss, xs, xd, ea, offs, w1a, w1c, b1r, W2, b2r,
                    We1[0], be1[0].reshape(1, 3 * NS), We2[0],
                    be2[0].reshape(1, NS), wcat[0], r16, gmat,
                    pa, pb, ca, cb, cc, j16, j32)
    p0, dacc = _sc_scatter_deg(di2, m0)
    h1, degc = _combine0(p0, dacc)
    hs1, hd1 = _sc_gather_h(h1, si2, di2)
    m1 = _edge1(es, hs1, hd1, We1[1], be1[1].reshape(1, 3 * NS), We2[1],
                be2[1].reshape(1, NS), wcat[1], r32, gmat)
    p1 = _sc_scatter(di2, m1)
    h2 = _combine1(p1, degc)
    return h2[:N_NODES]
